# baseline (speedup 1.0000x reference)
import jax
import jax.numpy as jnp
from jax.experimental import pallas as pl
from jax.experimental.pallas import tpu as pltpu

BATCH = 16384
MAX_LEN = 150
ROWS_PER_BLOCK = 2048


def _body(x_ref, ids_ref, mask_ref, type_ref):
    x = x_ref[...]
    ids_ref[...] = x
    mask_ref[...] = jnp.where(x == 0, 0, 1).astype(jnp.int32)
    type_ref[...] = jnp.zeros_like(x)


def kernel(inputs):
    grid = (BATCH // ROWS_PER_BLOCK,)
    spec = pl.BlockSpec((ROWS_PER_BLOCK, MAX_LEN), lambda i: (i, 0))
    out_shape = jax.ShapeDtypeStruct((BATCH, MAX_LEN), jnp.int32)
    ids, mask, type_ids = pl.pallas_call(
        _body,
        grid=grid,
        in_specs=[spec],
        out_specs=[spec, spec, spec],
        out_shape=[out_shape, out_shape, out_shape],
        compiler_params=pltpu.CompilerParams(
            dimension_semantics=("arbitrary",),
        ),
    )(inputs)
    return (ids, mask, type_ids)
